# baseline (speedup 1.0000x reference)
import functools

import jax
import jax.numpy as jnp
from jax.experimental import pallas as pl
from jax.experimental.pallas import tpu as pltpu

N_HEADS = 4
K_EXPERTS = 2
N_EXPERTS = 8
D_IN = 768
D_OUT = 32
HID = 128
N_TOK = 32768

TILE = 2048


def _prep_body(scaling_ref, Wb_ref, bb_ref, W1h_ref, b1_ref,
               M_ref, c_ref, loss_ref):
    s = scaling_ref[...]
    eio = jax.lax.broadcasted_iota(jnp.int32, (N_HEADS, N_EXPERTS), 1)
    m1 = jnp.max(s, axis=1, keepdims=True)
    idx1 = jnp.min(jnp.where(s == m1, eio, N_EXPERTS),
                   axis=1, keepdims=True)
    masked = jnp.where(eio == idx1, -jnp.inf, s)
    m2 = jnp.max(masked, axis=1, keepdims=True)
    idx2 = jnp.min(jnp.where(masked == m2, eio, N_EXPERTS),
                   axis=1, keepdims=True)
    e2 = jnp.exp(m2 - m1)
    denom = 1.0 + e2
    p1 = 1.0 / denom
    p2 = e2 / denom
    scT = (p1 * (eio == idx1).astype(jnp.float32)
           + p2 * (eio == idx2).astype(jnp.float32))
    gram = jax.lax.dot_general(scT, scT, (((1,), (1,)), ((), ())),
                               preferred_element_type=jnp.float32)
    r_io = jax.lax.broadcasted_iota(jnp.int32, (N_HEADS, N_HEADS), 0)
    c_io = jax.lax.broadcasted_iota(jnp.int32, (N_HEADS, N_HEADS), 1)
    gram = gram - (r_io == c_io).astype(jnp.float32)
    loss_ref[...] = jnp.sum(gram * gram, axis=(0, 1), keepdims=True)

    M = jnp.zeros((D_IN, HID), dtype=jnp.float32)
    c = b1_ref[...]
    for e in range(N_EXPERTS):
        A_e = jnp.zeros((D_OUT, HID), dtype=jnp.float32)
        for h in range(N_HEADS):
            A_e = A_e + scT[h:h + 1, e:e + 1] * W1h_ref[h]
        M = M + jax.lax.dot_general(Wb_ref[e], A_e,
                                    (((1,), (0,)), ((), ())),
                                    preferred_element_type=jnp.float32)
        c = c + jax.lax.dot_general(bb_ref[e:e + 1, :], A_e,
                                    (((1,), (0,)), ((), ())),
                                    preferred_element_type=jnp.float32)
    M_ref[...] = M
    c_ref[...] = c


def _main_body(M_ref, c_ref, WoT_ref, bo_ref, x_ref, out_ref):
    xt = x_ref[...]
    z = jax.lax.dot_general(xt, M_ref[...], (((1,), (0,)), ((), ())),
                            preferred_element_type=jnp.float32) + c_ref[...]
    hidden = jnp.maximum(z, 0.0) + jnp.log1p(jnp.exp(-jnp.abs(z)))
    outT = jax.lax.dot_general(WoT_ref[...], hidden,
                               (((1,), (1,)), ((), ())),
                               preferred_element_type=jnp.float32)
    out_ref[0] = outT + bo_ref[...]


@jax.jit
def _run(x, scaling, Wb, bb, W1h, b1, WoT, bo):
    n = x.shape[0]
    nblk = n // TILE
    M, c, loss = pl.pallas_call(
        _prep_body,
        in_specs=[
            pl.BlockSpec((N_HEADS, N_EXPERTS), lambda: (0, 0)),
            pl.BlockSpec((N_EXPERTS, D_IN, D_OUT), lambda: (0, 0, 0)),
            pl.BlockSpec((N_EXPERTS, D_OUT), lambda: (0, 0)),
            pl.BlockSpec((N_HEADS, D_OUT, HID), lambda: (0, 0, 0)),
            pl.BlockSpec((1, HID), lambda: (0, 0)),
        ],
        out_specs=[
            pl.BlockSpec((D_IN, HID), lambda: (0, 0)),
            pl.BlockSpec((1, HID), lambda: (0, 0)),
            pl.BlockSpec((1, 1), lambda: (0, 0)),
        ],
        out_shape=[
            jax.ShapeDtypeStruct((D_IN, HID), jnp.float32),
            jax.ShapeDtypeStruct((1, HID), jnp.float32),
            jax.ShapeDtypeStruct((1, 1), jnp.float32),
        ],
    )(scaling, Wb, bb, W1h, b1)

    out3 = pl.pallas_call(
        _main_body,
        grid=(nblk,),
        in_specs=[
            pl.BlockSpec((D_IN, HID), lambda i: (0, 0)),
            pl.BlockSpec((1, HID), lambda i: (0, 0)),
            pl.BlockSpec((1, HID), lambda i: (0, 0)),
            pl.BlockSpec((1, 1), lambda i: (0, 0)),
            pl.BlockSpec((TILE, D_IN), lambda i: (i, 0)),
        ],
        out_specs=pl.BlockSpec((1, 1, TILE), lambda i: (i, 0, 0)),
        out_shape=jax.ShapeDtypeStruct((nblk, 1, TILE), jnp.float32),
        compiler_params=pltpu.CompilerParams(
            dimension_semantics=("arbitrary",)),
    )(M, c, WoT, bo, x)
    return out3.reshape(n, 1), loss[0, 0]


def kernel(x, scaling, Wb, bb, W1, b1, Wo, bo):
    W1h = W1.reshape(D_OUT, N_HEADS, HID).transpose(1, 0, 2)
    return _run(x, scaling, Wb, bb, W1h, b1.reshape(1, HID),
                Wo.reshape(1, HID), bo.reshape(1, 1))


# speedup vs baseline: 1.1153x; 1.1153x over previous
import jax
import jax.numpy as jnp
from jax.experimental import pallas as pl
from jax.experimental.pallas import tpu as pltpu

N_HEADS = 4
K_EXPERTS = 2
N_EXPERTS = 8
D_IN = 768
D_OUT = 32
HID = 128
N_TOK = 32768

TILE = 4096


def _body(scaling_ref, Wb_ref, bb_ref, W1h_ref, b1_ref, WoT_ref, bo_ref,
          x_ref, out_ref, loss_ref, M_ref, c_ref):
    i = pl.program_id(0)

    @pl.when(i == 0)
    def _prologue():
        s = scaling_ref[...]
        eio = jax.lax.broadcasted_iota(jnp.int32, (N_HEADS, N_EXPERTS), 1)
        m1 = jnp.max(s, axis=1, keepdims=True)
        idx1 = jnp.min(jnp.where(s == m1, eio, N_EXPERTS),
                       axis=1, keepdims=True)
        masked = jnp.where(eio == idx1, -jnp.inf, s)
        m2 = jnp.max(masked, axis=1, keepdims=True)
        idx2 = jnp.min(jnp.where(masked == m2, eio, N_EXPERTS),
                       axis=1, keepdims=True)
        e2 = jnp.exp(m2 - m1)
        denom = 1.0 + e2
        p1 = 1.0 / denom
        p2 = e2 / denom
        scT = (p1 * (eio == idx1).astype(jnp.float32)
               + p2 * (eio == idx2).astype(jnp.float32))
        gram = jax.lax.dot_general(scT, scT, (((1,), (1,)), ((), ())),
                                   preferred_element_type=jnp.float32)
        r_io = jax.lax.broadcasted_iota(jnp.int32, (N_HEADS, N_HEADS), 0)
        c_io = jax.lax.broadcasted_iota(jnp.int32, (N_HEADS, N_HEADS), 1)
        gram = gram - (r_io == c_io).astype(jnp.float32)
        loss_ref[...] = jnp.sum(gram * gram, axis=(0, 1), keepdims=True)

        M = jnp.zeros((D_IN, HID), dtype=jnp.float32)
        c = b1_ref[...]
        for e in range(N_EXPERTS):
            A_e = jnp.zeros((D_OUT, HID), dtype=jnp.float32)
            for h in range(N_HEADS):
                A_e = A_e + scT[h:h + 1, e:e + 1] * W1h_ref[h]
            M = M + jax.lax.dot_general(Wb_ref[e], A_e,
                                        (((1,), (0,)), ((), ())),
                                        preferred_element_type=jnp.float32)
            c = c + jax.lax.dot_general(bb_ref[e:e + 1, :], A_e,
                                        (((1,), (0,)), ((), ())),
                                        preferred_element_type=jnp.float32)
        M_ref[...] = M
        c_ref[...] = c

    xt = x_ref[...]
    z = jax.lax.dot_general(xt, M_ref[...], (((1,), (0,)), ((), ())),
                            preferred_element_type=jnp.float32) + c_ref[...]
    hidden = jnp.maximum(z, 0.0) + jnp.log1p(jnp.exp(-jnp.abs(z)))
    outT = jax.lax.dot_general(WoT_ref[...], hidden,
                               (((1,), (1,)), ((), ())),
                               preferred_element_type=jnp.float32)
    out_ref[0] = outT + bo_ref[...]


@jax.jit
def _run(x, scaling, Wb, bb, W1h, b1, WoT, bo):
    n = x.shape[0]
    nblk = n // TILE
    out3, loss = pl.pallas_call(
        _body,
        grid=(nblk,),
        in_specs=[
            pl.BlockSpec((N_HEADS, N_EXPERTS), lambda i: (0, 0)),
            pl.BlockSpec((N_EXPERTS, D_IN, D_OUT), lambda i: (0, 0, 0)),
            pl.BlockSpec((N_EXPERTS, D_OUT), lambda i: (0, 0)),
            pl.BlockSpec((N_HEADS, D_OUT, HID), lambda i: (0, 0, 0)),
            pl.BlockSpec((1, HID), lambda i: (0, 0)),
            pl.BlockSpec((1, HID), lambda i: (0, 0)),
            pl.BlockSpec((1, 1), lambda i: (0, 0)),
            pl.BlockSpec((TILE, D_IN), lambda i: (i, 0)),
        ],
        out_specs=[
            pl.BlockSpec((1, 1, TILE), lambda i: (i, 0, 0)),
            pl.BlockSpec((1, 1), lambda i: (0, 0)),
        ],
        out_shape=[
            jax.ShapeDtypeStruct((nblk, 1, TILE), jnp.float32),
            jax.ShapeDtypeStruct((1, 1), jnp.float32),
        ],
        scratch_shapes=[
            pltpu.VMEM((D_IN, HID), jnp.float32),
            pltpu.VMEM((1, HID), jnp.float32),
        ],
        compiler_params=pltpu.CompilerParams(
            dimension_semantics=("arbitrary",)),
    )(scaling, Wb, bb, W1h, b1, WoT, bo, x)
    return out3.reshape(n, 1), loss[0, 0]


def kernel(x, scaling, Wb, bb, W1, b1, Wo, bo):
    W1h = W1.reshape(D_OUT, N_HEADS, HID).transpose(1, 0, 2)
    return _run(x, scaling, Wb, bb, W1h, b1.reshape(1, HID),
                Wo.reshape(1, HID), bo.reshape(1, 1))
